# Initial kernel scaffold; baseline (speedup 1.0000x reference)
import jax
import jax.numpy as jnp
from jax.experimental import pallas as pl


def kernel(x, w1, b1, w2, b2):
    raise NotImplementedError("write your pallas kernel here")


# speedup vs baseline: 2.0437x; 2.0437x over previous
import functools

import jax
import jax.numpy as jnp
from jax.experimental import pallas as pl
from jax.experimental.pallas import tpu as pltpu

_TILE = 10000
_CHUNK = 1000


def _round_up(n, m):
    return ((n + m - 1) // m) * m


def _mlp_body(x_ref, w1_ref, b1_ref, w2_ref, b2_ref, o_ref, *, tile, chunk):
    w1 = w1_ref[...]
    b1 = b1_ref[...]
    w2 = w2_ref[...]
    b2 = b2_ref[...]
    for c in range(0, tile, chunk):
        xs = x_ref[pl.ds(c, chunk), :]
        h = jnp.dot(xs, w1, preferred_element_type=jnp.float32)
        h = jnp.maximum(h + b1, 0.0)
        y = jnp.dot(h, w2, preferred_element_type=jnp.float32)
        o_ref[pl.ds(c, chunk), :] = (y + b2).astype(o_ref.dtype)


def kernel(x, w1, b1, w2, b2):
    batch, in_f = x.shape
    hid = w1.shape[1]
    out_f = w2.shape[1]
    dtype = x.dtype

    tile = min(_TILE, _round_up(batch, 8))
    chunk = min(_CHUNK, tile)
    if tile % chunk:
        tile = _round_up(tile, chunk)
    b_pad = _round_up(batch, tile)
    x_in = x if b_pad == batch else jnp.pad(x, ((0, b_pad - batch), (0, 0)))

    y = pl.pallas_call(
        functools.partial(_mlp_body, tile=tile, chunk=chunk),
        out_shape=jax.ShapeDtypeStruct((b_pad, out_f), dtype),
        grid=(b_pad // tile,),
        in_specs=[
            pl.BlockSpec((tile, in_f), lambda i: (i, 0)),
            pl.BlockSpec((in_f, hid), lambda i: (0, 0)),
            pl.BlockSpec((1, hid), lambda i: (0, 0)),
            pl.BlockSpec((hid, out_f), lambda i: (0, 0)),
            pl.BlockSpec((1, out_f), lambda i: (0, 0)),
        ],
        out_specs=pl.BlockSpec((tile, out_f), lambda i: (i, 0)),
        compiler_params=pltpu.CompilerParams(
            dimension_semantics=("parallel",)),
    )(x_in, w1, b1, w2, b2)
    return y if b_pad == batch else y[:batch]


# speedup vs baseline: 6.7567x; 3.3061x over previous
import functools

import jax
import jax.numpy as jnp
from jax.experimental import pallas as pl
from jax.experimental.pallas import tpu as pltpu

_BLOCK_COLS = 25600
_CHUNK_COLS = 3200


def _mlp_t_body(xt_ref, w1t_ref, b1t_ref, w2t_ref, b2t_ref, o_ref, *,
                block_cols, chunk_cols):
    w1t = w1t_ref[...]
    b1t = b1t_ref[...]
    w2t = w2t_ref[...]
    b2t = b2t_ref[...]
    for c in range(block_cols // chunk_cols):
        xs = xt_ref[:, pl.ds(c * chunk_cols, chunk_cols)]
        h = jnp.dot(w1t, xs, preferred_element_type=jnp.float32)
        h = jnp.maximum(h + b1t, 0.0)
        yt = jnp.dot(w2t, h, preferred_element_type=jnp.float32)
        o_ref[:, pl.ds(c * chunk_cols, chunk_cols)] = (
            yt + b2t).astype(o_ref.dtype)


def kernel(x, w1, b1, w2, b2):
    batch, in_f = x.shape
    hid = w1.shape[1]
    out_f = w2.shape[1]
    dtype = x.dtype

    xt = x.T
    w1t = w1.T
    b1t = b1.T
    w2t = w2.T
    b2t = b2.T

    block = min(_BLOCK_COLS, ((batch + 127) // 128) * 128)
    chunk = min(_CHUNK_COLS, block)
    if block % chunk:
        block = ((block + chunk - 1) // chunk) * chunk
    grid = -(-batch // block)

    yt = pl.pallas_call(
        functools.partial(_mlp_t_body, block_cols=block, chunk_cols=chunk),
        out_shape=jax.ShapeDtypeStruct((out_f, grid * block), dtype),
        grid=(grid,),
        in_specs=[
            pl.BlockSpec((in_f, block), lambda i: (0, i)),
            pl.BlockSpec((hid, in_f), lambda i: (0, 0)),
            pl.BlockSpec((hid, 1), lambda i: (0, 0)),
            pl.BlockSpec((out_f, hid), lambda i: (0, 0)),
            pl.BlockSpec((out_f, 1), lambda i: (0, 0)),
        ],
        out_specs=pl.BlockSpec((out_f, block), lambda i: (0, i)),
        compiler_params=pltpu.CompilerParams(
            dimension_semantics=("parallel",)),
    )(xt, w1t, b1t, w2t, b2t)
    return yt[:, :batch].T


# speedup vs baseline: 8.6380x; 1.2784x over previous
import functools

import jax
import jax.numpy as jnp
from jax.experimental import pallas as pl
from jax.experimental.pallas import tpu as pltpu

_BLOCK_COLS = 38400
_CHUNK_COLS = 3200


def _mlp_t_body(xt_ref, w1_ref, b1_ref, w2_ref, b2_ref, o_ref, *,
                block_cols, chunk_cols):
    w1t = jnp.concatenate(
        [w1_ref[...], b1_ref[...]], axis=0).T
    w2t = w2_ref[...].T
    b2t = b2_ref[...].T
    ones = jnp.ones((1, chunk_cols), xt_ref.dtype)
    for c in range(block_cols // chunk_cols):
        xs = xt_ref[:, pl.ds(c * chunk_cols, chunk_cols)]
        xa = jnp.concatenate([xs, ones], axis=0)
        h = jnp.dot(w1t, xa, preferred_element_type=jnp.float32)
        h = jnp.maximum(h, 0.0)
        yt = jnp.dot(w2t, h, preferred_element_type=jnp.float32)
        o_ref[:, pl.ds(c * chunk_cols, chunk_cols)] = (
            yt + b2t).astype(o_ref.dtype)


def kernel(x, w1, b1, w2, b2):
    batch, in_f = x.shape
    hid = w1.shape[1]
    out_f = w2.shape[1]
    dtype = x.dtype

    xt = x.T

    block = min(_BLOCK_COLS, ((batch + 127) // 128) * 128)
    chunk = min(_CHUNK_COLS, block)
    if block % chunk:
        block = ((block + chunk - 1) // chunk) * chunk
    grid = -(-batch // block)

    yt = pl.pallas_call(
        functools.partial(_mlp_t_body, block_cols=block, chunk_cols=chunk),
        out_shape=jax.ShapeDtypeStruct((out_f, batch), dtype),
        grid=(grid,),
        in_specs=[
            pl.BlockSpec((in_f, block), lambda i: (0, i)),
            pl.BlockSpec((in_f, hid), lambda i: (0, 0)),
            pl.BlockSpec((1, hid), lambda i: (0, 0)),
            pl.BlockSpec((hid, out_f), lambda i: (0, 0)),
            pl.BlockSpec((1, out_f), lambda i: (0, 0)),
        ],
        out_specs=pl.BlockSpec((out_f, block), lambda i: (0, i)),
        compiler_params=pltpu.CompilerParams(
            dimension_semantics=("arbitrary",)),
    )(xt, w1, b1, w2, b2)
    return yt.T


# speedup vs baseline: 8.6893x; 1.0059x over previous
import functools

import jax
import jax.numpy as jnp
from jax.experimental import pallas as pl
from jax.experimental.pallas import tpu as pltpu

_BLOCK_COLS = 38400
_CHUNK_COLS = 3200


def _mlp_t_body(xt_ref, w1_ref, b1_ref, w2_ref, b2_ref, o_ref, *,
                block_cols, chunk_cols):
    w1t = jnp.concatenate(
        [w1_ref[...], b1_ref[...]], axis=0).T
    w2t = w2_ref[...]
    b2t = b2_ref[...].T
    ones = jnp.ones((1, chunk_cols), xt_ref.dtype)
    for c in range(block_cols // chunk_cols):
        xs = xt_ref[:, pl.ds(c * chunk_cols, chunk_cols)]
        xa = jnp.concatenate([xs, ones], axis=0)
        h = jnp.dot(w1t, xa, preferred_element_type=jnp.float32)
        h = jnp.maximum(h, 0.0)
        yt = jnp.dot(w2t, h, preferred_element_type=jnp.float32)
        o_ref[:, pl.ds(c * chunk_cols, chunk_cols)] = (
            yt + b2t).astype(o_ref.dtype)


def kernel(x, w1, b1, w2, b2):
    batch, in_f = x.shape
    hid = w1.shape[1]
    out_f = w2.shape[1]
    dtype = x.dtype

    xt = x.T
    w2t = w2.T

    block = min(_BLOCK_COLS, ((batch + 127) // 128) * 128)
    chunk = min(_CHUNK_COLS, block)
    if block % chunk:
        block = ((block + chunk - 1) // chunk) * chunk
    grid = -(-batch // block)

    yt = pl.pallas_call(
        functools.partial(_mlp_t_body, block_cols=block, chunk_cols=chunk),
        out_shape=jax.ShapeDtypeStruct((out_f, batch), dtype),
        grid=(grid,),
        in_specs=[
            pl.BlockSpec((in_f, block), lambda i: (0, i)),
            pl.BlockSpec((in_f, hid), lambda i: (0, 0)),
            pl.BlockSpec((1, hid), lambda i: (0, 0)),
            pl.BlockSpec((out_f, hid), lambda i: (0, 0)),
            pl.BlockSpec((1, out_f), lambda i: (0, 0)),
        ],
        out_specs=pl.BlockSpec((out_f, block), lambda i: (0, i)),
        compiler_params=pltpu.CompilerParams(
            dimension_semantics=("arbitrary",)),
    )(xt, w1, b1, w2t, b2)
    return yt.T
